# Initial kernel scaffold; baseline (speedup 1.0000x reference)
import jax
import jax.numpy as jnp
from jax.experimental import pallas as pl


def kernel(features, W0, W1, W2, bn_w0, bn_b0, bn_w1, bn_b1, alpha, beta, gamma, delta, edge_index, edge_index_high):
    raise NotImplementedError("write your pallas kernel here")


# speedup vs baseline: 2.7673x; 2.7673x over previous
import jax
import jax.numpy as jnp
from jax import lax
from jax.experimental import pallas as pl
from jax.experimental.pallas import tpu as pltpu
from jax.experimental.pallas import tpu_sc as plsc

N = 10000
E = 320000
D = 128
NC = 2
NS = 16
LANES = 128
KI = 16
ROWS_ALL = 2560
E_PAD = ROWS_ALL * LANES
N_PAD = 10240
CP = N_PAD // NS

_f32 = jnp.float32
_i32 = jnp.int32

_MESH = plsc.VectorSubcoreMesh(
    core_axis_name="c", subcore_axis_name="s", num_cores=NC, num_subcores=NS)


def _sweep(t_hbm, src2, dst2, acc, sidxb, didxb, rows0, rows1, sem0, sem1,
           row0, n_outer):
    rowbufs = (rows0, rows1)
    sems = (sem0, sem1)

    @pl.loop(0, n_outer)
    def _(ko):
        base = row0 + ko * KI
        pltpu.sync_copy(src2.at[pl.ds(base, KI)], sidxb)
        pltpu.sync_copy(dst2.at[pl.ds(base, KI)], didxb)
        descs = [None, None]
        descs[0] = pltpu.async_copy(t_hbm.at[sidxb.at[0]], rowbufs[0], sems[0])
        for j in range(KI):
            b = j & 1
            descs[b].wait()
            if j + 1 < KI:
                nb = (j + 1) & 1
                descs[nb] = pltpu.async_copy(
                    t_hbm.at[sidxb.at[j + 1]], rowbufs[nb], sems[nb])
            pltpu.sync_copy(rowbufs[b], acc.at[didxb.at[j]], add=True)


def _ac_body(tg, tgh, sg2, dg2, sh2, dh2, z128, outa, outc,
             sidxb, didxb, rows0, rows1, acc, sem0, sem1):
    cid = lax.axis_index("c")
    sid = lax.axis_index("s")

    @pl.when(sid == 0)
    def _():
        pltpu.sync_copy(z128, acc)
    plsc.subcore_barrier()

    row0 = sid * (ROWS_ALL // NS)
    n_outer = ROWS_ALL // NS // KI

    @pl.when(cid == 0)
    def _():
        _sweep(tg, sg2, dg2, acc, sidxb, didxb, rows0, rows1, sem0, sem1,
               row0, n_outer)

    @pl.when(cid == 1)
    def _():
        _sweep(tgh, sh2, dh2, acc, sidxb, didxb, rows0, rows1, sem0, sem1,
               row0, n_outer)

    plsc.subcore_barrier()
    cp0 = sid * CP

    @pl.when(cid == 0)
    def _():
        pltpu.sync_copy(acc.at[pl.ds(cp0, CP)], outa.at[pl.ds(cp0, CP)])

    @pl.when(cid == 1)
    def _():
        pltpu.sync_copy(acc.at[pl.ds(cp0, CP)], outc.at[pl.ds(cp0, CP)])


def _d_body(td, sh2, dh2, z128, outd,
            sidxb, didxb, rows0, rows1, acc, sem0, sem1):
    cid = lax.axis_index("c")
    sid = lax.axis_index("s")

    @pl.when(sid == 0)
    def _():
        pltpu.sync_copy(z128, acc)
    plsc.subcore_barrier()

    per_core = ROWS_ALL // NC
    row0 = cid * per_core + sid * (per_core // NS)
    n_outer = per_core // NS // KI
    _sweep(td, sh2, dh2, acc, sidxb, didxb, rows0, rows1, sem0, sem1,
           row0, n_outer)

    plsc.subcore_barrier()
    cp0 = sid * CP
    pltpu.sync_copy(acc.at[pl.ds(cp0, CP)], outd.at[cid].at[pl.ds(cp0, CP)])


def _deg_body(dg2, dh2, z128, ones128, outdeg, didxb, onesb, acc):
    cid = lax.axis_index("c")
    sid = lax.axis_index("s")

    @pl.when(sid == 0)
    def _():
        pltpu.sync_copy(z128, acc)
    pltpu.sync_copy(ones128, onesb)
    plsc.subcore_barrier()

    row0 = sid * (ROWS_ALL // NS)
    n_outer = ROWS_ALL // NS // KI

    def _count(dst2):
        @pl.loop(0, n_outer)
        def _(ko):
            base = row0 + ko * KI
            pltpu.sync_copy(dst2.at[pl.ds(base, KI)], didxb)
            for j in range(KI):
                pltpu.sync_copy(onesb, acc.at[didxb.at[j]], add=True)

    @pl.when(cid == 0)
    def _():
        _count(dg2)

    @pl.when(cid == 1)
    def _():
        _count(dh2)

    plsc.subcore_barrier()
    cp0 = sid * CP
    pltpu.sync_copy(acc.at[pl.ds(cp0, CP)], outdeg.at[cid].at[pl.ds(cp0, CP)])


_deg = pl.kernel(
    _deg_body,
    out_type=jax.ShapeDtypeStruct((NC, N_PAD, D), _f32),
    mesh=_MESH,
    scratch_types=[
        pltpu.VMEM((KI, LANES), _i32),
        pltpu.VMEM((LANES, D), _f32),
        pltpu.VMEM_SHARED((N_PAD, D), _f32),
    ],
)

_ac = pl.kernel(
    _ac_body,
    out_type=(jax.ShapeDtypeStruct((N_PAD, D), _f32),
              jax.ShapeDtypeStruct((N_PAD, D), _f32)),
    mesh=_MESH,
    scratch_types=[
        pltpu.VMEM((KI, LANES), _i32),
        pltpu.VMEM((KI, LANES), _i32),
        pltpu.VMEM((LANES, D), _f32),
        pltpu.VMEM((LANES, D), _f32),
        pltpu.VMEM_SHARED((N_PAD, D), _f32),
        pltpu.SemaphoreType.DMA,
        pltpu.SemaphoreType.DMA,
    ],
)

_d = pl.kernel(
    _d_body,
    out_type=jax.ShapeDtypeStruct((NC, N_PAD, D), _f32),
    mesh=_MESH,
    scratch_types=[
        pltpu.VMEM((KI, LANES), _i32),
        pltpu.VMEM((KI, LANES), _i32),
        pltpu.VMEM((LANES, D), _f32),
        pltpu.VMEM((LANES, D), _f32),
        pltpu.VMEM_SHARED((N_PAD, D), _f32),
        pltpu.SemaphoreType.DMA,
        pltpu.SemaphoreType.DMA,
    ],
)


def _l2n(x):
    n = jnp.sqrt(jnp.sum(x * x, axis=1, keepdims=True))
    return x / jnp.maximum(n, 1e-12)


def _prep0_body(x_ref, wt_ref, deg_ref, hw_ref, tg_ref, tgh_ref,
                ng_ref, nh_ref):
    h = _l2n(x_ref[...])
    hw = lax.dot_general(h, wt_ref[...], (((1,), (0,)), ((), ())),
                         preferred_element_type=_f32)
    ng = lax.rsqrt(jnp.maximum(deg_ref[0, 0:N, 0:1], 1.0))
    nh = lax.rsqrt(jnp.maximum(deg_ref[1, 0:N, 0:1], 1.0))
    hw_ref[...] = hw
    tg_ref[...] = hw * ng
    tgh_ref[...] = hw * nh
    ng_ref[...] = ng
    nh_ref[...] = nh


_prep0 = pl.pallas_call(
    _prep0_body,
    out_shape=(jax.ShapeDtypeStruct((N, D), _f32),
               jax.ShapeDtypeStruct((N, D), _f32),
               jax.ShapeDtypeStruct((N, D), _f32),
               jax.ShapeDtypeStruct((N, 1), _f32),
               jax.ShapeDtypeStruct((N, 1), _f32)),
)


def _mid_body(acca_ref, ng_ref, nh_ref, td_ref):
    td_ref[...] = acca_ref[0:N, :] * (ng_ref[...] * nh_ref[...])


_mid = pl.pallas_call(
    _mid_body,
    out_shape=jax.ShapeDtypeStruct((N, D), _f32),
)


def _l2s_body(accc_ref, accd_ref, cn_ref, dn_ref):
    cn_ref[...] = _l2n(accc_ref[0:N, :])
    dn_ref[...] = _l2n(accd_ref[0, 0:N, :] + accd_ref[1, 0:N, :])


_l2s = pl.pallas_call(
    _l2s_body,
    out_shape=(jax.ShapeDtypeStruct((N, D), _f32),
               jax.ShapeDtypeStruct((N, D), _f32)),
)


def _comb_body(acca_ref, cn_ref, dn_ref, hw_ref, ng_ref, scal_ref, h_ref):
    scal = scal_ref[...]
    al = scal[0:1, 0:1]
    be = scal[0:1, 1:2]
    ga = scal[0:1, 2:3]
    de = scal[0:1, 3:4]
    h_ref[...] = (be * (acca_ref[0:N, :] * ng_ref[...]) - ga * cn_ref[...]
                  + al * hw_ref[...] - de * dn_ref[...])


_comb = pl.pallas_call(
    _comb_body,
    out_shape=jax.ShapeDtypeStruct((N, D), _f32),
)


def _prep_body(h_ref, wt_ref, bnw_ref, bnb_ref, ng_ref, nh_ref,
               hw2_ref, tg2_ref, tgh2_ref):
    h = h_ref[...]
    m = jnp.mean(h, axis=0, keepdims=True)
    v = jnp.mean((h - m) * (h - m), axis=0, keepdims=True)
    h = (h - m) / jnp.sqrt(v + 1e-5) * bnw_ref[...] + bnb_ref[...]
    h = jnp.maximum(h, 0.0)
    hw2 = lax.dot_general(h, wt_ref[...], (((1,), (0,)), ((), ())),
                          preferred_element_type=_f32)
    hw2_ref[...] = hw2
    tg2_ref[...] = hw2 * ng_ref[...]
    tgh2_ref[...] = hw2 * nh_ref[...]


_prep = pl.pallas_call(
    _prep_body,
    out_shape=(jax.ShapeDtypeStruct((N, D), _f32),
               jax.ShapeDtypeStruct((N, D), _f32),
               jax.ShapeDtypeStruct((N, D), _f32)),
)


def kernel(features, W0, W1, W2, bn_w0, bn_b0, bn_w1, bn_b1,
           alpha, beta, gamma, delta, edge_index, edge_index_high):
    padi = jnp.zeros((E_PAD - E,), _i32)
    padd = jnp.full((E_PAD - E,), N, _i32)
    sg2 = jnp.concatenate([edge_index[0], padi]).reshape(ROWS_ALL, LANES)
    dg2 = jnp.concatenate([edge_index[1], padd]).reshape(ROWS_ALL, LANES)
    sh2 = jnp.concatenate([edge_index_high[0], padi]).reshape(ROWS_ALL, LANES)
    dh2 = jnp.concatenate([edge_index_high[1], padd]).reshape(ROWS_ALL, LANES)
    z128 = jnp.zeros((N_PAD, D), _f32)
    ones128 = jnp.ones((LANES, D), _f32)
    scal = jnp.stack([alpha, beta, gamma, delta]).reshape(1, 4).astype(_f32)

    deg2 = _deg(dg2, dh2, z128, ones128)
    hw, tg, tgh, ng, nh = _prep0(features, W0.T, deg2)
    h = None
    for i in range(3):
        acca, accc = _ac(tg, tgh, sg2, dg2, sh2, dh2, z128)
        td = _mid(acca, ng, nh)
        accd = _d(td, sh2, dh2, z128)
        cn, dn = _l2s(accc, accd)
        h = _comb(acca, cn, dn, hw, ng, scal)
        if i < 2:
            bw = (bn_w0, bn_w1)[i].reshape(1, D)
            bb = (bn_b0, bn_b1)[i].reshape(1, D)
            wn = (W1, W2)[i]
            hw, tg, tgh = _prep(h, wn.T, bw, bb, ng, nh)
    return h
